# baseline (speedup 1.0000x reference)
import functools

import jax
import jax.numpy as jnp
from jax import lax
from jax.experimental import pallas as pl
from jax.experimental.pallas import tpu as pltpu
from jax.experimental.pallas import tpu_sc as plsc

_K = 8
_B = 4096
_NC = 2
_NS = 16
_NW = _NC * _NS
_BPW = _B // _NW
_H2 = _BPW * _K

_MESH = plsc.VectorSubcoreMesh(core_axis_name="c", subcore_axis_name="s")
_PARAMS = pltpu.CompilerParams(use_tc_tiling_on_sc=False)


@functools.partial(
    pl.kernel,
    mesh=_MESH,
    compiler_params=_PARAMS,
    out_type=[
        jax.ShapeDtypeStruct((_B, _K), jnp.int32),
        jax.ShapeDtypeStruct((_K, _B, _K), jnp.int32),
        jax.ShapeDtypeStruct((_B, _K), jnp.int32),
        jax.ShapeDtypeStruct((_K, _B, _K), jnp.int32),
    ],
    scratch_types=[
        pltpu.VMEM((_BPW,), jnp.int32),
        pltpu.VMEM((_BPW, _K), jnp.int32),
        pltpu.VMEM((_BPW, _K), jnp.int32),
        pltpu.VMEM((_K, _BPW), jnp.int32),
        pltpu.VMEM((_K, _BPW, _K), jnp.int32),
        pltpu.VMEM((_K, _BPW, _K), jnp.int32),
        pltpu.SemaphoreType.DMA,
        pltpu.SemaphoreType.DMA,
        pltpu.SemaphoreType.DMA,
    ],
)
def _encode(ent_hbm, adj_e_hbm, adj_eT_hbm, adj_r_hbm,
            ent1_hbm, ent2_hbm, rel1_hbm, rel2_hbm,
            idx_v, e1_v, r1_v, ecol_v, e2_v, r2_v, sem_a, sem_b, sem_w):
    wid = lax.axis_index("s") * _NC + lax.axis_index("c")
    base = pl.multiple_of(wid * _BPW, 8)

    pltpu.sync_copy(ent_hbm.at[pl.ds(base, _BPW)], idx_v)

    c_e1 = pltpu.async_copy(adj_e_hbm.at[idx_v], e1_v, sem_a)
    c_r1 = pltpu.async_copy(adj_r_hbm.at[idx_v], r1_v, sem_b)
    cols = [
        pltpu.async_copy(adj_eT_hbm.at[j].at[idx_v], ecol_v.at[j], sem_a)
        for j in range(_K)
    ]
    for cp in cols:
        cp.wait()

    h2 = []
    for j in range(_K):
        ic = ecol_v.at[j]
        h2.append(pltpu.async_copy(adj_e_hbm.at[ic], e2_v.at[j], sem_a))
        h2.append(pltpu.async_copy(adj_r_hbm.at[ic], r2_v.at[j], sem_b))

    c_e1.wait()
    w_e1 = pltpu.async_copy(e1_v, ent1_hbm.at[pl.ds(base, _BPW)], sem_w)
    c_r1.wait()
    w_r1 = pltpu.async_copy(r1_v, rel1_hbm.at[pl.ds(base, _BPW)], sem_w)

    wb2 = []
    for cp in h2:
        cp.wait()
    for j in range(_K):
        wb2.append(pltpu.async_copy(
            e2_v.at[j], ent2_hbm.at[j, pl.ds(base, _BPW)], sem_w))
        wb2.append(pltpu.async_copy(
            r2_v.at[j], rel2_hbm.at[j, pl.ds(base, _BPW)], sem_w))

    w_e1.wait()
    w_r1.wait()
    for cp in wb2:
        cp.wait()


def kernel(entity, adj_entity, adj_relation):
    ent1, ent2, rel1, rel2 = _encode(
        entity.reshape(-1), adj_entity, adj_entity.T, adj_relation)
    return (
        entity,
        ent1,
        ent2.transpose(1, 0, 2).reshape(_B, _K * _K),
        rel1,
        rel2.transpose(1, 0, 2).reshape(_B, _K * _K),
    )


# speedup vs baseline: 1.2374x; 1.2374x over previous
import functools

import jax
import jax.numpy as jnp
from jax import lax
from jax.experimental import pallas as pl
from jax.experimental.pallas import tpu as pltpu
from jax.experimental.pallas import tpu_sc as plsc

_K = 8
_B = 4096
_L = 16
_NC = 2
_NS = 16
_NW = _NC * _NS
_BPW = _B // _NW
_H2 = _BPW * _K

_MESH = plsc.VectorSubcoreMesh(core_axis_name="c", subcore_axis_name="s")
_PARAMS = pltpu.CompilerParams(use_tc_tiling_on_sc=False)


@functools.partial(
    pl.kernel,
    mesh=_MESH,
    compiler_params=_PARAMS,
    out_type=[
        jax.ShapeDtypeStruct((_B * _K,), jnp.int32),
        jax.ShapeDtypeStruct((_B * _K, _K), jnp.int32),
        jax.ShapeDtypeStruct((_B, _K), jnp.int32),
        jax.ShapeDtypeStruct((_B * _K, _K), jnp.int32),
    ],
    scratch_types=[
        pltpu.VMEM((_BPW,), jnp.int32),
        pltpu.VMEM((_H2,), jnp.int32),
        pltpu.VMEM((_H2,), jnp.int32),
        pltpu.VMEM((_BPW, _K), jnp.int32),
        pltpu.VMEM((_H2, _K), jnp.int32),
        pltpu.VMEM((_H2, _K), jnp.int32),
        pltpu.SemaphoreType.DMA,
        pltpu.SemaphoreType.DMA,
        pltpu.SemaphoreType.DMA,
    ],
)
def _encode(ent_hbm, adj_e_hbm, adj_r_hbm, adj_ef_hbm,
            ent1_hbm, ent2_hbm, rel1_hbm, rel2_hbm,
            idx_v, eidx_v, e1_v, r1_v, e2_v, r2_v, sem_a, sem_b, sem_w):
    wid = lax.axis_index("s") * _NC + lax.axis_index("c")
    base = pl.multiple_of(wid * _BPW, 8)
    base2 = pl.multiple_of(wid * _H2, 8)

    pltpu.sync_copy(ent_hbm.at[pl.ds(base, _BPW)], idx_v)

    c_r1 = pltpu.async_copy(adj_r_hbm.at[idx_v], r1_v, sem_b)

    lanes = lax.iota(jnp.int32, _L)
    sub = lanes >> 3
    offs = lanes & 7
    for tt in range(_BPW // _L):
        e16 = idx_v[pl.ds(tt * _L, _L)]
        for v in range(_L // 2):
            vec = jnp.where(sub == 0, e16[2 * v], e16[2 * v + 1])
            eidx_v[pl.ds(tt * (_L * _K) + v * _L, _L)] = vec * _K + offs

    h1 = [
        pltpu.async_copy(
            adj_ef_hbm.at[eidx_v.at[pl.ds(c * 128, 128)]],
            e1_v.at[pl.ds(c * 128, 128)], sem_a)
        for c in range(_K)
    ]
    for cp in h1:
        cp.wait()

    h2 = []
    for c in range(_K):
        off = c * 128
        ic = e1_v.at[pl.ds(off, 128)]
        h2.append(pltpu.async_copy(
            adj_e_hbm.at[ic], e2_v.at[pl.ds(off, 128)], sem_a))
        h2.append(pltpu.async_copy(
            adj_r_hbm.at[ic], r2_v.at[pl.ds(off, 128)], sem_b))

    w_e1 = pltpu.async_copy(e1_v, ent1_hbm.at[pl.ds(base2, _H2)], sem_w)
    c_r1.wait()
    w_r1 = pltpu.async_copy(r1_v, rel1_hbm.at[pl.ds(base, _BPW)], sem_w)

    for cp in h2:
        cp.wait()
    w_e2 = pltpu.async_copy(e2_v, ent2_hbm.at[pl.ds(base2, _H2)], sem_w)
    w_r2 = pltpu.async_copy(r2_v, rel2_hbm.at[pl.ds(base2, _H2)], sem_w)

    w_e1.wait()
    w_r1.wait()
    w_e2.wait()
    w_r2.wait()


def kernel(entity, adj_entity, adj_relation):
    adj_ef = adj_entity.reshape(-1)
    adj_ef = adj_ef.at[0].set(adj_ef[0])
    ent1, ent2, rel1, rel2 = _encode(
        entity.reshape(-1), adj_entity, adj_relation, adj_ef)
    return (
        entity,
        ent1.reshape(_B, _K),
        ent2.reshape(_B, _K * _K),
        rel1,
        rel2.reshape(_B, _K * _K),
    )
